# baseline (speedup 1.0000x reference)
import functools

import jax
import jax.numpy as jnp
from jax import lax
from jax.experimental import pallas as pl
from jax.experimental.pallas import tpu as pltpu
from jax.experimental.pallas import tpu_sc as plsc

_NC = 2
_NS = 16
_NW = _NC * _NS
_L = 16


def _make_sc_kernel(B: int, F: int, R: int):
    rows_per_w = B // _NW
    chunks = rows_per_w // R
    blocks_per_col = R // _L
    mesh = plsc.VectorSubcoreMesh(core_axis_name="c", subcore_axis_name="s")

    @functools.partial(
        pl.kernel,
        out_type=jax.ShapeDtypeStruct((2 * F, B // 128, 128), jnp.float32),
        mesh=mesh,
        compiler_params=pltpu.CompilerParams(needs_layout_passes=False),
        scratch_types=[
            pltpu.VMEM((F, R), jnp.int32),
            pltpu.VMEM((F, R), jnp.int32),
            pltpu.VMEM((2 * F, 1, R), jnp.float32),
            pltpu.VMEM((2 * F, 1, R), jnp.float32),
            pltpu.VMEM((_L, _L), jnp.float32),
            pltpu.VMEM((_L, _L), jnp.float32),
            pltpu.VMEM((_L,), jnp.float32),
            pltpu.VMEM((_L,), jnp.float32),
            pltpu.VMEM((_L,), jnp.float32),
            pltpu.VMEM((_L,), jnp.float32),
            pltpu.SemaphoreType.DMA,
            pltpu.SemaphoreType.DMA,
        ],
    )
    def sc_kernel(x_hbm, t1_hbm, t2_hbm, w_hbm, b_hbm, out_hbm,
                  x0_v, x1_v, oa_v, ob_v,
                  t1_v, t2_v, w_v, b_v, lut1_v, lut2_v,
                  in_sem, out_sem):
        x_bufs = (x0_v, x1_v)
        o_bufs = (oa_v, ob_v)
        pltpu.sync_copy(t1_hbm, t1_v)
        pltpu.sync_copy(t2_hbm, t2_v)
        pltpu.sync_copy(w_hbm, w_v)
        pltpu.sync_copy(b_hbm, b_v)

        lanes = lax.iota(jnp.int32, _L)
        wv = w_v[...]
        acc1 = b_v[...]
        acc2 = b_v[...]
        for k in range(10):
            kk = jnp.full((_L,), k, jnp.int32)
            wk = jnp.sum(jnp.where(lanes == k, wv, 0.0))
            acc1 = acc1 + plsc.load_gather(t1_v, [lanes, kk]) * wk
            acc2 = acc2 + plsc.load_gather(t2_v, [lanes, kk]) * wk
        lut1_v[...] = acc1
        lut2_v[...] = acc2

        wid = lax.axis_index("s") * _NC + lax.axis_index("c")
        row0 = wid * rows_per_w

        def in_copy(c, buf):
            return pltpu.make_async_copy(
                x_hbm.at[:, pl.ds(row0 + c * R, R)], x_bufs[buf], in_sem)

        q0 = row0 // 128

        def out_copies(c):
            return (
                pltpu.make_async_copy(
                    o_bufs[c & 1], out_hbm.at[:, pl.ds(q0 + c, 1), :], out_sem),
            )

        in_copy(0, 0).start()
        for c in range(chunks):
            buf = c & 1
            in_copy(c, buf).wait()
            if c + 1 < chunks:
                in_copy(c + 1, buf ^ 1).start()
            if c >= 2:
                for cp in out_copies(c - 2):
                    cp.wait()
            x_c = x_bufs[buf]
            o_c = o_bufs[buf]

            @plsc.parallel_loop(0, F * blocks_per_col, unroll=8)
            def _(t):
                f = t // blocks_per_col
                rb = (t % blocks_per_col) * _L
                idx = x_c[f, pl.ds(rb, _L)]
                o_c[2 * f, 0, pl.ds(rb, _L)] = plsc.load_gather(lut1_v, [idx])
                o_c[2 * f + 1, 0, pl.ds(rb, _L)] = plsc.load_gather(lut2_v, [idx])

            for cp in out_copies(c):
                cp.start()

        for c in range(max(chunks - 2, 0), chunks):
            for cp in out_copies(c):
                cp.wait()

    return sc_kernel


@jax.jit
def kernel(x, table1, table2, W, b):
    B, F = x.shape
    t1p = jnp.zeros((_L, _L), jnp.float32).at[:10, :10].set(table1)
    t2p = jnp.zeros((_L, _L), jnp.float32).at[:10, :10].set(table2)
    wp = jnp.zeros((_L,), jnp.float32).at[:10].set(W[:, 0])
    bp = jnp.broadcast_to(b, (_L,))
    res = _make_sc_kernel(B, F, R=128)(x.T, t1p, t2p, wp, bp)
    return res.reshape(F, 2 * B).T.reshape(2 * B, F, 1)


# speedup vs baseline: 1.3246x; 1.3246x over previous
import functools

import jax
import jax.numpy as jnp
from jax import lax
from jax.experimental import pallas as pl
from jax.experimental.pallas import tpu as pltpu
from jax.experimental.pallas import tpu_sc as plsc

_NC = 2
_NS = 16
_NW = _NC * _NS
_L = 16


def _make_sc_kernel(B: int, F: int, R: int):
    rows_per_w = B // _NW
    chunks = rows_per_w // R
    blocks_per_col = R // _L
    mesh = plsc.VectorSubcoreMesh(core_axis_name="c", subcore_axis_name="s")

    @functools.partial(
        pl.kernel,
        out_type=jax.ShapeDtypeStruct((F, 2 * B), jnp.float32),
        mesh=mesh,
        compiler_params=pltpu.CompilerParams(needs_layout_passes=False),
        scratch_types=[
            pltpu.VMEM((F, R), jnp.int32),
            pltpu.VMEM((F, R), jnp.int32),
            pltpu.VMEM((F, R), jnp.float32),
            pltpu.VMEM((F, R), jnp.float32),
            pltpu.VMEM((F, R), jnp.float32),
            pltpu.VMEM((F, R), jnp.float32),
            pltpu.VMEM((_L, _L), jnp.float32),
            pltpu.VMEM((_L, _L), jnp.float32),
            pltpu.VMEM((_L,), jnp.float32),
            pltpu.VMEM((_L,), jnp.float32),
            pltpu.VMEM((_L,), jnp.float32),
            pltpu.VMEM((_L,), jnp.float32),
            pltpu.SemaphoreType.DMA,
            pltpu.SemaphoreType.DMA,
        ],
    )
    def sc_kernel(x_hbm, t1_hbm, t2_hbm, w_hbm, b_hbm, out_hbm,
                  x0_v, x1_v, o1a_v, o1b_v, o2a_v, o2b_v,
                  t1_v, t2_v, w_v, b_v, lut1_v, lut2_v,
                  in_sem, out_sem):
        x_bufs = (x0_v, x1_v)
        o1_bufs = (o1a_v, o1b_v)
        o2_bufs = (o2a_v, o2b_v)
        pltpu.sync_copy(t1_hbm, t1_v)
        pltpu.sync_copy(t2_hbm, t2_v)
        pltpu.sync_copy(w_hbm, w_v)
        pltpu.sync_copy(b_hbm, b_v)

        lanes = lax.iota(jnp.int32, _L)
        wv = w_v[...]
        acc1 = b_v[...]
        acc2 = b_v[...]
        for k in range(10):
            kk = jnp.full((_L,), k, jnp.int32)
            wk = jnp.sum(jnp.where(lanes == k, wv, 0.0))
            acc1 = acc1 + plsc.load_gather(t1_v, [lanes, kk]) * wk
            acc2 = acc2 + plsc.load_gather(t2_v, [lanes, kk]) * wk
        lut1_v[...] = acc1
        lut2_v[...] = acc2

        wid = lax.axis_index("s") * _NC + lax.axis_index("c")
        row0 = wid * rows_per_w

        def in_copy(c, buf):
            return pltpu.make_async_copy(
                x_hbm.at[:, pl.ds(row0 + c * R, R)], x_bufs[buf], in_sem)

        def out_copies(c):
            buf = c & 1
            return (
                pltpu.make_async_copy(
                    o1_bufs[buf], out_hbm.at[:, pl.ds(row0 + c * R, R)], out_sem),
                pltpu.make_async_copy(
                    o2_bufs[buf], out_hbm.at[:, pl.ds(B + row0 + c * R, R)], out_sem),
            )

        in_copy(0, 0).start()
        for c in range(chunks):
            buf = c & 1
            in_copy(c, buf).wait()
            if c + 1 < chunks:
                in_copy(c + 1, buf ^ 1).start()
            if c >= 2:
                for cp in out_copies(c - 2):
                    cp.wait()
            x_c = x_bufs[buf]
            o1_c = o1_bufs[buf]
            o2_c = o2_bufs[buf]

            @plsc.parallel_loop(0, F * blocks_per_col, unroll=16)
            def _(t):
                f = t // blocks_per_col
                rb = (t % blocks_per_col) * _L
                idx = x_c[f, pl.ds(rb, _L)]
                o1_c[f, pl.ds(rb, _L)] = plsc.load_gather(lut1_v, [idx])
                o2_c[f, pl.ds(rb, _L)] = plsc.load_gather(lut2_v, [idx])

            for cp in out_copies(c):
                cp.start()

        for c in range(max(chunks - 2, 0), chunks):
            for cp in out_copies(c):
                cp.wait()

    return sc_kernel


@jax.jit
def kernel(x, table1, table2, W, b):
    B, F = x.shape
    t1p = jnp.zeros((_L, _L), jnp.float32).at[:10, :10].set(table1)
    t2p = jnp.zeros((_L, _L), jnp.float32).at[:10, :10].set(table2)
    wp = jnp.zeros((_L,), jnp.float32).at[:10].set(W[:, 0])
    bp = jnp.broadcast_to(b, (_L,))
    out_fm = _make_sc_kernel(B, F, R=128)(x.T, t1p, t2p, wp, bp)
    return out_fm.T.reshape(2 * B, F, 1)


# speedup vs baseline: 1.3374x; 1.0096x over previous
import functools

import jax
import jax.numpy as jnp
from jax import lax
from jax.experimental import pallas as pl
from jax.experimental.pallas import tpu as pltpu
from jax.experimental.pallas import tpu_sc as plsc

_NC = 2
_NS = 16
_NW = _NC * _NS
_L = 16


def _make_sc_kernel(B: int, F: int, R: int):
    rows_per_w = B // _NW
    chunks = rows_per_w // R
    blocks_per_col = R // _L
    mesh = plsc.VectorSubcoreMesh(core_axis_name="c", subcore_axis_name="s")

    @functools.partial(
        pl.kernel,
        out_type=jax.ShapeDtypeStruct((F, 2 * B), jnp.float32),
        mesh=mesh,
        compiler_params=pltpu.CompilerParams(needs_layout_passes=False),
        scratch_types=[
            pltpu.VMEM((F, R), jnp.int32),
            pltpu.VMEM((F, R), jnp.int32),
            pltpu.VMEM((F, R), jnp.float32),
            pltpu.VMEM((F, R), jnp.float32),
            pltpu.VMEM((F, R), jnp.float32),
            pltpu.VMEM((F, R), jnp.float32),
            pltpu.VMEM((_L, _L), jnp.float32),
            pltpu.VMEM((_L, _L), jnp.float32),
            pltpu.VMEM((_L,), jnp.float32),
            pltpu.VMEM((_L,), jnp.float32),
            pltpu.VMEM((_L,), jnp.float32),
            pltpu.VMEM((_L,), jnp.float32),
            pltpu.SemaphoreType.DMA,
            pltpu.SemaphoreType.DMA,
        ],
    )
    def sc_kernel(x_hbm, t1_hbm, t2_hbm, w_hbm, b_hbm, out_hbm,
                  x0_v, x1_v, o1a_v, o1b_v, o2a_v, o2b_v,
                  t1_v, t2_v, w_v, b_v, lut1_v, lut2_v,
                  in_sem, out_sem):
        x_bufs = (x0_v, x1_v)
        o1_bufs = (o1a_v, o1b_v)
        o2_bufs = (o2a_v, o2b_v)
        pltpu.sync_copy(t1_hbm, t1_v)
        pltpu.sync_copy(t2_hbm, t2_v)
        pltpu.sync_copy(w_hbm, w_v)
        pltpu.sync_copy(b_hbm, b_v)

        lanes = lax.iota(jnp.int32, _L)
        wv = w_v[...]
        acc1 = b_v[...]
        acc2 = b_v[...]
        for k in range(10):
            kk = jnp.full((_L,), k, jnp.int32)
            wk = jnp.sum(jnp.where(lanes == k, wv, 0.0))
            acc1 = acc1 + plsc.load_gather(t1_v, [lanes, kk]) * wk
            acc2 = acc2 + plsc.load_gather(t2_v, [lanes, kk]) * wk
        lut1_v[...] = acc1
        lut2_v[...] = acc2

        wid = lax.axis_index("s") * _NC + lax.axis_index("c")
        row0 = wid * rows_per_w

        def in_copy(c, buf):
            return pltpu.make_async_copy(
                x_hbm.at[:, pl.ds(row0 + c * R, R)], x_bufs[buf], in_sem)

        def out_copies(c):
            buf = c & 1
            return (
                pltpu.make_async_copy(
                    o1_bufs[buf], out_hbm.at[:, pl.ds(row0 + c * R, R)], out_sem),
                pltpu.make_async_copy(
                    o2_bufs[buf], out_hbm.at[:, pl.ds(B + row0 + c * R, R)], out_sem),
            )

        in_copy(0, 0).start()
        for c in range(chunks):
            buf = c & 1
            in_copy(c, buf).wait()
            if c + 1 < chunks:
                in_copy(c + 1, buf ^ 1).start()
            if c >= 2:
                for cp in out_copies(c - 2):
                    cp.wait()
            x_c = x_bufs[buf]
            o1_c = o1_bufs[buf]
            o2_c = o2_bufs[buf]

            @plsc.parallel_loop(0, F * blocks_per_col, unroll=8)
            def _(t):
                f = t // blocks_per_col
                rb = (t % blocks_per_col) * _L
                idx = x_c[f, pl.ds(rb, _L)]
                o1_c[f, pl.ds(rb, _L)] = plsc.load_gather(lut1_v, [idx])
                o2_c[f, pl.ds(rb, _L)] = plsc.load_gather(lut2_v, [idx])

            for cp in out_copies(c):
                cp.start()

        for c in range(max(chunks - 2, 0), chunks):
            for cp in out_copies(c):
                cp.wait()

    return sc_kernel


@jax.jit
def kernel(x, table1, table2, W, b):
    B, F = x.shape
    t1p = jnp.zeros((_L, _L), jnp.float32).at[:10, :10].set(table1)
    t2p = jnp.zeros((_L, _L), jnp.float32).at[:10, :10].set(table2)
    wp = jnp.zeros((_L,), jnp.float32).at[:10].set(W[:, 0])
    bp = jnp.broadcast_to(b, (_L,))
    out_fm = _make_sc_kernel(B, F, R=128)(x.T, t1p, t2p, wp, bp)
    return out_fm.T.reshape(2 * B, F, 1)
